# baseline (speedup 1.0000x reference)
import functools

import jax
import jax.numpy as jnp
from jax import lax
from jax.experimental import pallas as pl
from jax.experimental.pallas import tpu as pltpu
from jax.experimental.pallas import tpu_sc as plsc

N = 10000
D = 256
E = 160000
NC = 2
NS = 16
HALF = D // 2
CHUNK = 125
CHUNKS_PER_TILE = E // (NS * CHUNK)
HALF_CHUNKS = CHUNKS_PER_TILE // 2
ZROWS = 40
WROWS = 80
BIG_TILE_ROWS = 640
LAST_TILE_ROWS = N - 15 * BIG_TILE_ROWS


def _sc_segment_sum(r, ys_and_edges):
    mesh = plsc.VectorSubcoreMesh(
        core_axis_name="c", subcore_axis_name="s",
        num_cores=NC, num_subcores=NS)
    n_types = len(ys_and_edges)
    flat_ops = []
    for y, s_, d_ in ys_and_edges:
        flat_ops += [y, s_, d_]

    @functools.partial(
        pl.kernel,
        out_type=jax.ShapeDtypeStruct((N, NC, HALF), jnp.float32),
        mesh=mesh,
        scratch_types=[
            pltpu.VMEM_SHARED((N, 1, HALF), jnp.float32),
            pltpu.VMEM((HALF_CHUNKS, CHUNK), jnp.int32),
            pltpu.VMEM((HALF_CHUNKS, CHUNK), jnp.int32),
            pltpu.VMEM((CHUNK, 1, HALF), jnp.float32),
            pltpu.VMEM((CHUNK, 1, HALF), jnp.float32),
            pltpu.SemaphoreType.DMA,
            pltpu.SemaphoreType.DMA,
            pltpu.SemaphoreType.DMA,
            pltpu.SemaphoreType.DMA,
            pltpu.SemaphoreType.DMA,
        ],
    )
    def seg_kernel(r_hbm, *refs):
        y_hbms = [refs[3 * t] for t in range(n_types)]
        src_hbms = [refs[3 * t + 1] for t in range(n_types)]
        dst_hbms = [refs[3 * t + 2] for t in range(n_types)]
        (out_hbm, acc, idx_s, idx_d, rows0, rows1,
         gsem0, gsem1, ssem0, ssem1, zwsem) = refs[3 * n_types:]
        c = lax.axis_index("c")
        s = lax.axis_index("s")
        row_base = s * BIG_TILE_ROWS
        n_zdma = jnp.where(s < NS - 1, BIG_TILE_ROWS // ZROWS,
                           LAST_TILE_ROWS // ZROWS)
        n_wdma = jnp.where(s < NS - 1, BIG_TILE_ROWS // WROWS,
                           LAST_TILE_ROWS // WROWS)

        @pl.loop(0, n_zdma)
        def _(i):
            r0 = row_base + i * ZROWS
            pltpu.async_copy(r_hbm.at[pl.ds(r0, ZROWS), pl.ds(c, 1)],
                             acc.at[pl.ds(r0, ZROWS)], zwsem)

        @pl.loop(0, n_zdma)
        def _(i):
            pltpu.make_async_copy(
                r_hbm.at[pl.ds(row_base, ZROWS), pl.ds(c, 1)],
                acc.at[pl.ds(row_base, ZROWS)], zwsem).wait()
        plsc.subcore_barrier()

        def one_type(x_hbm, src_hbm, dst_hbm):
            def g_start(j, buf, sem):
                pltpu.async_copy(x_hbm.at[idx_s.at[j], pl.ds(c, 1)], buf, sem)

            def g_wait(buf, sem):
                pltpu.make_async_copy(
                    x_hbm.at[idx_s.at[0], pl.ds(c, 1)], buf, sem).wait()

            def s_start(j, buf, sem):
                pltpu.async_copy(buf, acc.at[idx_d.at[j]], sem, add=True)

            def s_wait(j, buf, sem):
                pltpu.make_async_copy(buf, acc.at[idx_d.at[j]], sem).wait()

            for h in range(2):
                ch0 = s * CHUNKS_PER_TILE + h * HALF_CHUNKS
                pltpu.sync_copy(src_hbm.at[pl.ds(ch0, HALF_CHUNKS)], idx_s)
                pltpu.sync_copy(dst_hbm.at[pl.ds(ch0, HALF_CHUNKS)], idx_d)
                npair = HALF_CHUNKS // 2
                g_start(0, rows0, gsem0)

                @pl.loop(0, npair)
                def _(p):
                    j0 = 2 * p
                    j1 = j0 + 1
                    g_wait(rows0, gsem0)

                    @pl.when(p > 0)
                    def _():
                        s_wait(j1, rows1, ssem1)
                    g_start(j1, rows1, gsem1)
                    s_start(j0, rows0, ssem0)
                    g_wait(rows1, gsem1)
                    s_wait(j0, rows0, ssem0)

                    @pl.when(p < npair - 1)
                    def _():
                        g_start(j0 + 2, rows0, gsem0)
                    s_start(j1, rows1, ssem1)

                s_wait(HALF_CHUNKS - 1, rows1, ssem1)

        for t in range(n_types):
            one_type(y_hbms[t], src_hbms[t], dst_hbms[t])
        plsc.subcore_barrier()

        @pl.loop(0, n_wdma)
        def _(i):
            r0 = row_base + i * WROWS
            pltpu.async_copy(acc.at[pl.ds(r0, WROWS)],
                             out_hbm.at[pl.ds(r0, WROWS), pl.ds(c, 1)], zwsem)

        @pl.loop(0, n_wdma)
        def _(i):
            pltpu.make_async_copy(
                acc.at[pl.ds(row_base, WROWS)],
                out_hbm.at[pl.ds(row_base, WROWS), pl.ds(c, 1)], zwsem).wait()
        plsc.subcore_barrier()

    return seg_kernel(r, *flat_ops)


def _tc_matmuls(xs, ws, n_out):
    BM = 2000

    def mm_kernel(*refs):
        ins = refs[:2 * n_out]
        outs = refs[2 * n_out:]
        dot = functools.partial(lax.dot, preferred_element_type=jnp.float32)
        for i in range(n_out):
            outs[i][...] = dot(ins[i][...], ins[n_out + i][...])

    x_spec = pl.BlockSpec((BM, D), lambda i: (i, 0))
    w_spec = pl.BlockSpec((D, D), lambda i: (0, 0))
    return pl.pallas_call(
        mm_kernel,
        grid=(N // BM,),
        in_specs=[x_spec] * n_out + [w_spec] * n_out,
        out_specs=[x_spec] * n_out,
        out_shape=[jax.ShapeDtypeStruct((N, D), jnp.float32)] * n_out,
    )(*xs, *ws)


def kernel(x_a, x_b, edge_index_ab, edge_index_bb, edge_index_ba,
           W_rel_ab, W_root_ab, W_rel_bb, W_root_bb, W_rel_ba, W_root_ba):
    def prep(ei):
        ei = ei.astype(jnp.int32)
        src2 = ei[0].reshape(E // CHUNK, CHUNK)
        dst2 = ei[1].reshape(E // CHUNK, CHUNK)
        return src2, dst2

    sab, dab = prep(edge_index_ab)
    sbb, dbb = prep(edge_index_bb)
    sba, dba = prep(edge_index_ba)

    def half_view(t):
        return t.reshape(N, NC, HALF)

    y_ba, r_a = _tc_matmuls([x_b, x_a], [W_rel_ba, W_root_ba], 2)
    out_a = _sc_segment_sum(half_view(r_a), [(half_view(y_ba), sba, dba)])
    y_ab, y_bb, r_b = _tc_matmuls(
        [x_a, x_b, x_b], [W_rel_ab, W_rel_bb, W_root_ab + W_root_bb], 3)
    out_b = _sc_segment_sum(
        half_view(r_b),
        [(half_view(y_ab), sab, dab), (half_view(y_bb), sbb, dbb)])
    return (out_a.reshape(N, D), out_b.reshape(N, D))


# speedup vs baseline: 1.1963x; 1.1963x over previous
import functools

import jax
import jax.numpy as jnp
from jax import lax
from jax.experimental import pallas as pl
from jax.experimental.pallas import tpu as pltpu
from jax.experimental.pallas import tpu_sc as plsc

N = 10000
D = 256
E = 160000
NC = 2
NS = 16
HALF = D // 2
CHUNK = 125
CHUNKS_PER_TILE = E // (NS * CHUNK)
HALF_CHUNKS = CHUNKS_PER_TILE // 2
ZROWS = 40
WROWS = 80
BIG_TILE_ROWS = 640
LAST_TILE_ROWS = N - 15 * BIG_TILE_ROWS


def _sc_segment_sums(x_cat_a, x_cat_b, sab, dab, sbb, dbb, sba, dba):
    mesh = plsc.VectorSubcoreMesh(
        core_axis_name="c", subcore_axis_name="s",
        num_cores=NC, num_subcores=NS)
    agg_t = jax.ShapeDtypeStruct((NC, N, 1, HALF), jnp.float32)

    @functools.partial(
        pl.kernel,
        out_type=(agg_t, agg_t, agg_t),
        mesh=mesh,
        scratch_types=[
            pltpu.VMEM_SHARED((N, 1, HALF), jnp.float32),
            pltpu.VMEM((HALF_CHUNKS, CHUNK), jnp.int32),
            pltpu.VMEM((HALF_CHUNKS, CHUNK), jnp.int32),
            pltpu.VMEM((CHUNK, 1, HALF), jnp.float32),
            pltpu.VMEM((CHUNK, 1, HALF), jnp.float32),
            pltpu.VMEM((ZROWS, 1, HALF), jnp.float32),
            pltpu.SemaphoreType.DMA,
            pltpu.SemaphoreType.DMA,
            pltpu.SemaphoreType.DMA,
            pltpu.SemaphoreType.DMA,
            pltpu.SemaphoreType.DMA,
        ],
    )
    def seg_kernel(xa_hbm, xb_hbm, sab_h, dab_h, sbb_h, dbb_h, sba_h, dba_h,
                   out_ab, out_bb, out_ba, acc, idx_s, idx_d, rows0, rows1,
                   zbuf, gsem0, gsem1, ssem0, ssem1, zwsem):
        c = lax.axis_index("c")
        s = lax.axis_index("s")
        row_base = s * BIG_TILE_ROWS
        n_zdma = jnp.where(s < NS - 1, BIG_TILE_ROWS // ZROWS,
                           LAST_TILE_ROWS // ZROWS)
        n_wdma = jnp.where(s < NS - 1, BIG_TILE_ROWS // WROWS,
                           LAST_TILE_ROWS // WROWS)

        @pl.loop(0, ZROWS)
        def _(i):
            @pl.loop(0, HALF // 16)
            def _(j):
                zbuf[i, 0, pl.ds(j * 16, 16)] = jnp.zeros((16,), jnp.float32)

        def one_type(x_hbm, src_hbm, dst_hbm, out_hbm):
            @pl.loop(0, n_zdma)
            def _(i):
                pltpu.async_copy(
                    zbuf, acc.at[pl.ds(row_base + i * ZROWS, ZROWS)], zwsem)

            @pl.loop(0, n_zdma)
            def _(i):
                pltpu.make_async_copy(
                    zbuf, acc.at[pl.ds(row_base, ZROWS)], zwsem).wait()
            plsc.subcore_barrier()

            def g_start(j, buf, sem):
                pltpu.async_copy(x_hbm.at[idx_s.at[j], pl.ds(c, 1)], buf, sem)

            def g_wait(buf, sem):
                pltpu.make_async_copy(x_hbm.at[idx_s.at[0], pl.ds(c, 1)], buf, sem).wait()

            def s_start(j, buf, sem):
                pltpu.async_copy(buf, acc.at[idx_d.at[j]], sem, add=True)

            def s_wait(j, buf, sem):
                pltpu.make_async_copy(buf, acc.at[idx_d.at[j]], sem).wait()

            for h in range(2):
                ch0 = s * CHUNKS_PER_TILE + h * HALF_CHUNKS
                pltpu.sync_copy(src_hbm.at[pl.ds(ch0, HALF_CHUNKS)], idx_s)
                pltpu.sync_copy(dst_hbm.at[pl.ds(ch0, HALF_CHUNKS)], idx_d)
                npair = HALF_CHUNKS // 2
                g_start(0, rows0, gsem0)

                @pl.loop(0, npair)
                def _(p):
                    j0 = 2 * p
                    j1 = j0 + 1

                    @pl.when(p > 0)
                    def _():
                        s_wait(j1, rows1, ssem1)
                    g_start(j1, rows1, gsem1)
                    g_wait(rows0, gsem0)
                    s_start(j0, rows0, ssem0)
                    g_wait(rows1, gsem1)
                    s_wait(j0, rows0, ssem0)

                    @pl.when(p < npair - 1)
                    def _():
                        g_start(j0 + 2, rows0, gsem0)
                    s_start(j1, rows1, ssem1)

                s_wait(HALF_CHUNKS - 1, rows1, ssem1)
            plsc.subcore_barrier()

            @pl.loop(0, n_wdma)
            def _(i):
                r0 = row_base + i * WROWS
                pltpu.async_copy(acc.at[pl.ds(r0, WROWS)],
                                 out_hbm.at[c, pl.ds(r0, WROWS)], zwsem)

            @pl.loop(0, n_wdma)
            def _(i):
                pltpu.make_async_copy(
                    acc.at[pl.ds(row_base, WROWS)],
                    out_hbm.at[c, pl.ds(row_base, WROWS)], zwsem).wait()
            plsc.subcore_barrier()

        one_type(xa_hbm, sab_h, dab_h, out_ab)
        one_type(xb_hbm, sbb_h, dbb_h, out_bb)
        one_type(xb_hbm, sba_h, dba_h, out_ba)

    return seg_kernel(x_cat_a, x_cat_b, sab, dab, sbb, dbb, sba, dba)


def _tc_combine(agg_ab, agg_bb, agg_ba, x_a, x_b,
                W_rel_ab, W_root_ab, W_rel_bb, W_root_bb, W_rel_ba, W_root_ba):
    BM = 2000

    def mm_kernel(aab, abb, aba, xa, xb, wab, wrab, wbb, wrbb, wba, wrba,
                  oa, ob):
        dot = functools.partial(lax.dot, preferred_element_type=jnp.float32)
        wab_ = wab[...]
        wbb_ = wbb[...]
        wba_ = wba[...]
        ob[...] = (dot(aab[0], wab_[:HALF]) + dot(aab[1], wab_[HALF:])
                   + dot(abb[0], wbb_[:HALF]) + dot(abb[1], wbb_[HALF:])
                   + dot(xb[...], wrab[...] + wrbb[...]))
        oa[...] = (dot(aba[0], wba_[:HALF]) + dot(aba[1], wba_[HALF:])
                   + dot(xa[...], wrba[...]))

    agg_spec = pl.BlockSpec((NC, BM, HALF), lambda i: (0, i, 0))
    x_spec = pl.BlockSpec((BM, D), lambda i: (i, 0))
    w_spec = pl.BlockSpec((D, D), lambda i: (0, 0))
    return pl.pallas_call(
        mm_kernel,
        grid=(N // BM,),
        in_specs=[agg_spec, agg_spec, agg_spec, x_spec, x_spec,
                  w_spec, w_spec, w_spec, w_spec, w_spec, w_spec],
        out_specs=[x_spec, x_spec],
        out_shape=[jax.ShapeDtypeStruct((N, D), jnp.float32)] * 2,
    )(agg_ab, agg_bb, agg_ba, x_a, x_b,
      W_rel_ab, W_root_ab, W_rel_bb, W_root_bb, W_rel_ba, W_root_ba)


def kernel(x_a, x_b, edge_index_ab, edge_index_bb, edge_index_ba,
           W_rel_ab, W_root_ab, W_rel_bb, W_root_bb, W_rel_ba, W_root_ba):
    def prep(ei):
        ei = ei.astype(jnp.int32)
        src2 = ei[0].reshape(E // CHUNK, CHUNK)
        dst2 = ei[1].reshape(E // CHUNK, CHUNK)
        return src2, dst2

    sab, dab = prep(edge_index_ab)
    sbb, dbb = prep(edge_index_bb)
    sba, dba = prep(edge_index_ba)
    x_cat_a = x_a.reshape(N, NC, HALF)
    x_cat_b = x_b.reshape(N, NC, HALF)

    agg_ab, agg_bb, agg_ba = _sc_segment_sums(
        x_cat_a, x_cat_b, sab, dab, sbb, dbb, sba, dba)
    agg_ab = agg_ab.reshape(NC, N, HALF)
    agg_bb = agg_bb.reshape(NC, N, HALF)
    agg_ba = agg_ba.reshape(NC, N, HALF)
    out_a, out_b = _tc_combine(
        agg_ab, agg_bb, agg_ba, x_a, x_b,
        W_rel_ab, W_root_ab, W_rel_bb, W_root_bb, W_rel_ba, W_root_ba)
    return (out_a, out_b)
